# baseline (speedup 1.0000x reference)
import jax
import jax.numpy as jnp
from jax import lax
from jax.experimental import pallas as pl
from jax.experimental.pallas import tpu as pltpu
from jax.experimental.pallas import tpu_sc as plsc

B = 16384
D = 64
CH = 128
NIDX = 6
NREL = 5

NC, NS, L = 2, 16, 16
NW = NC * NS
BPW = B // NW
NCHUNK = BPW // CH
DL = D // L


def _body(uid, wro, cit, coa, ven, aff,
          author_t, venue_t, affil_t, rel_t, doc_t,
          o_user, o_wrote, o_cited, o_coauth, o_venue, o_affil,
          o_r0, o_r1, o_r2, o_r3, o_r4,
          idx_all, rel_vmem, rel_blk, buf, tb0, tb1, isem, gsem, wsem, rsem):
    wid = lax.axis_index("s") * NC + lax.axis_index("c")
    base = wid * BPW

    idx_srcs = (uid, wro, cit, coa, ven, aff)
    tables = (author_t, doc_t, doc_t, author_t, venue_t, affil_t)
    outs = (o_user, o_wrote, o_cited, o_coauth, o_venue, o_affil)
    rel_outs = (o_r0, o_r1, o_r2, o_r3, o_r4)
    tbufs = (tb0, tb1)
    lanes = lax.iota(jnp.int32, L)

    idx_cps = [pltpu.async_copy(idx_srcs[g].at[wid], idx_all.at[g], isem)
               for g in range(NIDX)]

    pltpu.sync_copy(rel_t, rel_vmem)
    for r in range(NREL):
        rv = [rel_vmem[r, pl.ds(c * L, L)] for c in range(DL)]

        def rel_col(j, _, rv=rv):
            jv = jnp.full((L,), j, jnp.int32)
            for c in range(DL):
                plsc.store_scatter(rel_blk, [lanes + c * L, jv], rv[c])
            return 0

        lax.fori_loop(0, CH, rel_col, 0)
        cps = [
            pltpu.async_copy(rel_blk,
                             rel_outs[r].at[:, pl.ds(base + j * CH, CH)],
                             rsem)
            for j in range(BPW // CH)
        ]
        for c in cps:
            c.wait()

    for c in idx_cps:
        c.wait()

    wcps = [None] * NIDX
    for g in range(NIDX):
        slot = g % 2
        cps = [pltpu.async_copy(tables[g].at[idx_all.at[g, j]],
                                buf.at[pl.ds(j * CH, CH)], gsem)
               for j in range(NCHUNK)]
        for c in cps:
            c.wait()
        if g >= 2:
            wcps[g - 2].wait()
        tb = tbufs[slot]

        def tr(j, _, tb=tb):
            jv = jnp.full((L,), j, jnp.int32)
            for c in range(DL):
                rowv = buf[j, pl.ds(c * L, L)]
                plsc.store_scatter(tb, [lanes + c * L, jv], rowv)
            return 0

        lax.fori_loop(0, BPW, tr, 0)
        wcps[g] = pltpu.async_copy(tb, outs[g].at[:, pl.ds(base, BPW)], wsem)
    wcps[NIDX - 2].wait()
    wcps[NIDX - 1].wait()


@jax.jit
def _run(uid, wro, cit, coa, ven, aff, author_t, venue_t, affil_t, rel_t, doc_t):
    out = jax.ShapeDtypeStruct((D, B), jnp.float32)
    k = pl.kernel(
        _body,
        out_type=[out] * 11,
        mesh=plsc.VectorSubcoreMesh(core_axis_name="c", subcore_axis_name="s",
                                    num_cores=NC, num_subcores=NS),
        scratch_types=[
            pltpu.VMEM((NIDX, NCHUNK, CH), jnp.int32),
            pltpu.VMEM((NREL, D), jnp.float32),
            pltpu.VMEM((D, CH), jnp.float32),
            pltpu.VMEM((BPW, D), jnp.float32),
            pltpu.VMEM((D, BPW), jnp.float32),
            pltpu.VMEM((D, BPW), jnp.float32),
            pltpu.SemaphoreType.DMA,
            pltpu.SemaphoreType.DMA,
            pltpu.SemaphoreType.DMA,
            pltpu.SemaphoreType.DMA,
        ],
        compiler_params=pltpu.CompilerParams(use_tc_tiling_on_sc=False,
                                             needs_layout_passes=False),
    )
    return k(uid, wro, cit, coa, ven, aff,
             author_t, venue_t, affil_t, rel_t, doc_t)


def kernel(user_id, wrote, cited, coauthor, venue, affiliation,
           author_table, venue_table, affiliation_table, relation_table,
           doc_embs):
    def prep(i):
        return i.astype(jnp.int32).reshape(NW, NCHUNK, CH)

    res = _run(prep(user_id), prep(wrote), prep(cited), prep(coauthor),
               prep(venue), prep(affiliation),
               author_table, venue_table, affiliation_table, relation_table,
               doc_embs)
    return tuple(jnp.transpose(o) for o in res)


# speedup vs baseline: 1.1132x; 1.1132x over previous
import jax
import jax.numpy as jnp
from jax import lax
from jax.experimental import pallas as pl
from jax.experimental.pallas import tpu as pltpu
from jax.experimental.pallas import tpu_sc as plsc

B = 16384
D = 64
DP = 128
CH = 128
HT = 256
NIDX = 6
NREL = 5
NHALF = NIDX * 2

NC, NS, L = 2, 16, 16
NW = NC * NS
BPW = B // NW


def _body(uid, wro, cit, coa, ven, aff,
          author_t, venue_t, affil_t, rel_t, doc_t,
          o_user, o_wrote, o_cited, o_coauth, o_venue, o_affil,
          o_r0, o_r1, o_r2, o_r3, o_r4,
          idx_all, rel_vmem, rel_blk, buf0, buf1, isem, gsem, wsem, rsem):
    wid = lax.axis_index("s") * NC + lax.axis_index("c")
    base = wid * BPW

    idx_srcs = (uid, wro, cit, coa, ven, aff)
    tables = (author_t, doc_t, doc_t, author_t, venue_t, affil_t)
    outs = (o_user, o_wrote, o_cited, o_coauth, o_venue, o_affil)
    rel_outs = (o_r0, o_r1, o_r2, o_r3, o_r4)
    bufs = (buf0, buf1)

    idx_cps = [pltpu.async_copy(idx_srcs[g].at[pl.ds(base, BPW)],
                                idx_all.at[g], isem)
               for g in range(NIDX)]

    pltpu.sync_copy(rel_t, rel_vmem)
    rel_cps = []
    for r in range(NREL):
        rows = [rel_vmem[r, pl.ds(c * L, L)] for c in range(DP // L)]

        def rep(i, _, rows=rows):
            for c in range(DP // L):
                rel_blk[i, pl.ds(c * L, L)] = rows[c]
            return 0

        lax.fori_loop(0, CH, rep, 0)
        cps = [
            pltpu.async_copy(rel_blk,
                             rel_outs[r].at[pl.ds(base + j * CH, CH), :],
                             rsem)
            for j in range(BPW // CH)
        ]
        for c in cps:
            c.wait()
        rel_cps += cps

    for c in idx_cps:
        c.wait()

    wcps = [None] * NHALF
    for h in range(NHALF):
        g, half = divmod(h, 2)
        slot = h % 2
        if h >= 2:
            wcps[h - 2].wait()
        cps = [pltpu.async_copy(
                   tables[g].at[idx_all.at[g, pl.ds(half * HT + j * CH, CH)]],
                   bufs[slot].at[pl.ds(j * CH, CH)], gsem)
               for j in range(HT // CH)]
        for c in cps:
            c.wait()
        wcps[h] = pltpu.async_copy(
            bufs[slot], outs[g].at[pl.ds(base + half * HT, HT), :], wsem)
    wcps[NHALF - 2].wait()
    wcps[NHALF - 1].wait()


@jax.jit
def _run(uid, wro, cit, coa, ven, aff, author_t, venue_t, affil_t, rel_t, doc_t):
    out = jax.ShapeDtypeStruct((B, DP), jnp.float32)
    k = pl.kernel(
        _body,
        out_type=[out] * 11,
        mesh=plsc.VectorSubcoreMesh(core_axis_name="c", subcore_axis_name="s",
                                    num_cores=NC, num_subcores=NS),
        scratch_types=[
            pltpu.VMEM((NIDX, BPW), jnp.int32),
            pltpu.VMEM((NREL, DP), jnp.float32),
            pltpu.VMEM((CH, DP), jnp.float32),
            pltpu.VMEM((HT, DP), jnp.float32),
            pltpu.VMEM((HT, DP), jnp.float32),
            pltpu.SemaphoreType.DMA,
            pltpu.SemaphoreType.DMA,
            pltpu.SemaphoreType.DMA,
            pltpu.SemaphoreType.DMA,
        ],
        compiler_params=pltpu.CompilerParams(use_tc_tiling_on_sc=True),
    )
    res = k(uid, wro, cit, coa, ven, aff,
            author_t, venue_t, affil_t, rel_t, doc_t)
    return tuple(o[:, :D] for o in res)


def _pad(t):
    return jnp.pad(t, ((0, 0), (0, DP - D)))


def kernel(user_id, wrote, cited, coauthor, venue, affiliation,
           author_table, venue_table, affiliation_table, relation_table,
           doc_embs):
    return _run(user_id.astype(jnp.int32), wrote.astype(jnp.int32),
                cited.astype(jnp.int32), coauthor.astype(jnp.int32),
                venue.astype(jnp.int32), affiliation.astype(jnp.int32),
                _pad(author_table), _pad(venue_table),
                _pad(affiliation_table), _pad(relation_table),
                _pad(doc_embs))
